# baseline (speedup 1.0000x reference)
import functools

import jax
import jax.numpy as jnp
from jax.experimental import pallas as pl
from jax.experimental.pallas import tpu as pltpu
from jax.experimental.pallas import tpu_sc as plsc

NUM_ROWS = 48
WIDTH = 4096


@functools.partial(
    pl.kernel,
    out_type=jax.ShapeDtypeStruct((1, WIDTH), jnp.float32),
    mesh=plsc.ScalarSubcoreMesh(axis_name="c", num_cores=1),
    scratch_types=[pltpu.SMEM((1,), jnp.int32)],
)
def _gather_row(idx_hbm, table_hbm, out_hbm, idx_s):
    pltpu.sync_copy(idx_hbm, idx_s)
    row = idx_s[0]
    pltpu.sync_copy(table_hbm.at[pl.ds(row, 1)], out_hbm)


def kernel(layer, layer_embedding):
    idx = jnp.asarray(layer, jnp.int32).reshape(1)
    table = layer_embedding.reshape(NUM_ROWS, WIDTH)
    out = _gather_row(idx, table)
    return out.reshape(1, 1, WIDTH)
